# baseline (speedup 1.0000x reference)
import jax
import jax.numpy as jnp
from jax.experimental import pallas as pl

NUM_CLASSES = 100000
BLOCK_CLS = 4000


def _onehot_block(users_ref, out_ref):
    j = pl.program_id(0)
    rows = jax.lax.broadcasted_iota(jnp.int32, out_ref.shape, 0) + j * BLOCK_CLS
    out_ref[:, :] = (users_ref[:, :] == rows).astype(jnp.float32)


def kernel(users):
    b = users.shape[0]
    users2 = users.reshape(1, b)
    out_t = pl.pallas_call(
        _onehot_block,
        grid=(pl.cdiv(NUM_CLASSES, BLOCK_CLS),),
        in_specs=[pl.BlockSpec((1, b), lambda j: (0, 0))],
        out_specs=pl.BlockSpec((BLOCK_CLS, b), lambda j: (j, 0)),
        out_shape=jax.ShapeDtypeStruct((NUM_CLASSES, b), jnp.float32),
    )(users2)
    return out_t.T


# speedup vs baseline: 1.0313x; 1.0313x over previous
import jax
import jax.numpy as jnp
from jax.experimental import pallas as pl

NUM_CLASSES = 100000
BLOCK_CLS = 1000


def _onehot_block(users_ref, out_ref):
    j = pl.program_id(0)
    rows = jax.lax.broadcasted_iota(jnp.int32, out_ref.shape, 0) + j * BLOCK_CLS
    out_ref[:, :] = (users_ref[:, :] == rows).astype(jnp.float32)


def kernel(users):
    b = users.shape[0]
    users2 = users.reshape(1, b)
    out_t = pl.pallas_call(
        _onehot_block,
        grid=(pl.cdiv(NUM_CLASSES, BLOCK_CLS),),
        in_specs=[pl.BlockSpec((1, b), lambda j: (0, 0))],
        out_specs=pl.BlockSpec((BLOCK_CLS, b), lambda j: (j, 0)),
        out_shape=jax.ShapeDtypeStruct((NUM_CLASSES, b), jnp.float32),
    )(users2)
    return out_t.T
